# baseline (speedup 1.0000x reference)
import functools

import jax
import jax.numpy as jnp
from jax import lax
from jax.experimental import pallas as pl
from jax.experimental.pallas import tpu as pltpu
from jax.experimental.pallas import tpu_sc as plsc

N_CLASS = 1000000
EMBED_DIM = 32
BATCH = 16384
FIELDS = 26

_B = BATCH * FIELDS
_NC, _NS = 2, 16
_NW = _NC * _NS

_LANES = 16
_TW = 512
_NBLK = N_CLASS // _TW
_NBLK_REM = N_CLASS - _NBLK * _TW
_BLK_PER_W = _NBLK // _NW
_BLK_EXTRA = _NBLK % _NW

_mesh = plsc.VectorSubcoreMesh(core_axis_name="c", subcore_axis_name="s")


@functools.partial(
    pl.kernel,
    mesh=_mesh,
    compiler_params=pltpu.CompilerParams(
        use_tc_tiling_on_sc=True, needs_layout_passes=False
    ),
    out_type=jax.ShapeDtypeStruct((N_CLASS // 4, 128), jnp.float32),
    scratch_types=[
        pltpu.VMEM((2, EMBED_DIM, _TW), jnp.float32),
        pltpu.VMEM((2, _TW // 4, 128), jnp.float32),
        pltpu.SemaphoreType.DMA,
        pltpu.SemaphoreType.DMA,
    ],
)
def _transpose_kernel(table_t_hbm, tail_hbm, out_hbm, vblk, tblk, isem, osem):
    wid = lax.axis_index("s") * _NC + lax.axis_index("c")
    lo = wid * _BLK_PER_W + jnp.minimum(wid, _BLK_EXTRA)
    hi = lo + _BLK_PER_W + jnp.where(wid < _BLK_EXTRA, 1, 0)

    iota = lax.iota(jnp.int32, _LANES)
    colsbase = lax.rem(iota, 4) * EMBED_DIM
    rows_g = [16 * g // 4 + iota // 4 for g in range(_TW // _LANES)]

    def issue_in(b, buf):
        return pltpu.async_copy(
            table_t_hbm.at[:, pl.ds(b * _TW, _TW)], vblk.at[buf], isem
        )

    def issue_out(b, buf):
        return pltpu.async_copy(
            tblk.at[buf], out_hbm.at[pl.ds(b * (_TW // 4), _TW // 4)], osem
        )

    def compute(buf):
        @plsc.parallel_loop(0, EMBED_DIM, unroll=8)
        def drow(d):
            cols_d = colsbase + d
            for g in range(_TW // _LANES):
                vals = vblk[buf, d, pl.ds(16 * g, 16)]
                plsc.store_scatter(tblk.at[buf], [rows_g[g], cols_d], vals)

    issue_in(lo, 0)

    def body(k, carry):
        buf = lax.rem(k - lo, 2)
        pltpu.make_async_copy(
            table_t_hbm.at[:, pl.ds(k * _TW, _TW)], vblk.at[buf], isem
        ).wait()

        @pl.when(k + 1 < hi)
        def _():
            issue_in(k + 1, lax.rem(k + 1 - lo, 2))

        @pl.when(k - 2 >= lo)
        def _():
            pltpu.make_async_copy(
                tblk.at[buf],
                out_hbm.at[pl.ds((k - 2) * (_TW // 4), _TW // 4)],
                osem,
            ).wait()

        compute(buf)
        issue_out(k, buf)
        return carry

    lax.fori_loop(lo, hi, body, 0)

    def drain(k, carry):
        pltpu.make_async_copy(
            tblk.at[lax.rem(k - lo, 2)],
            out_hbm.at[pl.ds(k * (_TW // 4), _TW // 4)],
            osem,
        ).wait()
        return carry

    lax.fori_loop(jnp.maximum(lo, hi - 2), hi, drain, 0)

    @pl.when(wid == 0)
    def _():
        pltpu.sync_copy(tail_hbm, tblk.at[0, pl.ds(0, _NBLK_REM // 4)])
        pltpu.sync_copy(
            tblk.at[0, pl.ds(0, _NBLK_REM // 4)],
            out_hbm.at[pl.ds(_NBLK * (_TW // 4), _NBLK_REM // 4)],
        )


_BPW = _B // _NW
_CB = 32
_CHUNK = _CB * FIELDS
_NCHUNK = _BPW // _CHUNK
_BATCH_PER_W = BATCH // _NW


@functools.partial(
    pl.kernel,
    mesh=_mesh,
    compiler_params=pltpu.CompilerParams(
        use_tc_tiling_on_sc=False, needs_layout_passes=False
    ),
    out_type=jax.ShapeDtypeStruct((FIELDS, EMBED_DIM, BATCH), jnp.float32),
    scratch_types=[
        pltpu.VMEM((_BPW,), jnp.int32),
        pltpu.VMEM((2, _CHUNK, EMBED_DIM), jnp.float32),
        pltpu.VMEM((2, EMBED_DIM, _CHUNK), jnp.float32),
        pltpu.SemaphoreType.DMA,
        pltpu.SemaphoreType.DMA,
    ],
)
def _gather_kernel(idx_hbm, table_hbm, out_hbm, idx_v, rows_v, trans_v,
                   gsem, osem):
    wid = lax.axis_index("s") * _NC + lax.axis_index("c")
    b0 = wid * _BATCH_PER_W
    pltpu.sync_copy(idx_hbm.at[pl.ds(wid * _BPW, _BPW)], idx_v)

    iota = lax.iota(jnp.int32, _LANES)

    def issue_gather(c, buf):
        return pltpu.async_copy(
            table_hbm.at[idx_v.at[pl.ds(c * _CHUNK, _CHUNK)]],
            rows_v.at[buf], gsem,
        )

    def transpose_chunk(buf):
        @plsc.parallel_loop(0, EMBED_DIM, unroll=8)
        def drow(d):
            dvec = jnp.full((_LANES,), d, jnp.int32)
            for g in range(_CHUNK // _LANES):
                vals = plsc.load_gather(
                    rows_v.at[buf], [16 * g + iota, dvec]
                )
                trans_v[buf, d, pl.ds(16 * g, 16)] = vals

    def issue_stores(c, buf):
        for f in range(FIELDS):
            pltpu.async_copy(
                trans_v.at[buf, :, pl.ds(f * _CB, _CB)],
                out_hbm.at[f, :, pl.ds(b0 + c * _CB, _CB)],
                osem,
            )

    def wait_stores(c, buf):
        for f in range(FIELDS):
            pltpu.make_async_copy(
                trans_v.at[buf, :, pl.ds(f * _CB, _CB)],
                out_hbm.at[f, :, pl.ds(b0 + c * _CB, _CB)],
                osem,
            ).wait()

    issue_gather(0, 0)

    def body(c, carry):
        buf = lax.rem(c, 2)
        pltpu.make_async_copy(
            table_hbm.at[idx_v.at[pl.ds(c * _CHUNK, _CHUNK)]],
            rows_v.at[buf], gsem,
        ).wait()

        @pl.when(c + 1 < _NCHUNK)
        def _():
            issue_gather(c + 1, lax.rem(c + 1, 2))

        @pl.when(c >= 2)
        def _():
            wait_stores(c - 2, buf)

        transpose_chunk(buf)
        issue_stores(c, buf)
        return carry

    lax.fori_loop(0, _NCHUNK, body, 0)
    wait_stores(_NCHUNK - 2, lax.rem(_NCHUNK - 2, 2))
    wait_stores(_NCHUNK - 1, lax.rem(_NCHUNK - 1, 2))


def kernel(z_category, categ_embed_weight):
    idx = (
        z_category.reshape(_NW, _NCHUNK, _CB, FIELDS)
        .transpose(0, 1, 3, 2)
        .reshape(-1)
        .astype(jnp.int32)
    )
    tail = categ_embed_weight[_NBLK * _TW:].reshape(_NBLK_REM // 4, 128)
    table_packed = _transpose_kernel(categ_embed_weight.T, tail)
    table_rm = table_packed.reshape(N_CLASS, EMBED_DIM)
    out_t = _gather_kernel(idx, table_rm)
    return out_t.transpose(2, 0, 1)


# speedup vs baseline: 1.2561x; 1.2561x over previous
import functools

import jax
import jax.numpy as jnp
from jax import lax
from jax.experimental import pallas as pl
from jax.experimental.pallas import tpu as pltpu
from jax.experimental.pallas import tpu_sc as plsc

N_CLASS = 1000000
EMBED_DIM = 32
BATCH = 16384
FIELDS = 26

_B = BATCH * FIELDS
_NC, _NS = 2, 16
_NW = _NC * _NS

_LANES = 16
_TW = 512
_NBLK = N_CLASS // _TW
_NBLK_REM = N_CLASS - _NBLK * _TW
_BLK_PER_W = _NBLK // _NW
_BLK_EXTRA = _NBLK % _NW

_mesh = plsc.VectorSubcoreMesh(core_axis_name="c", subcore_axis_name="s")


@functools.partial(
    pl.kernel,
    mesh=_mesh,
    compiler_params=pltpu.CompilerParams(
        use_tc_tiling_on_sc=True, needs_layout_passes=False
    ),
    out_type=jax.ShapeDtypeStruct((N_CLASS // 4, 128), jnp.float32),
    scratch_types=[
        pltpu.VMEM((2, EMBED_DIM, _TW), jnp.float32),
        pltpu.VMEM((2, _TW // 4, 128), jnp.float32),
        pltpu.SemaphoreType.DMA,
        pltpu.SemaphoreType.DMA,
    ],
)
def _transpose_kernel(table_t_hbm, tail_hbm, out_hbm, vblk, tblk, isem, osem):
    wid = lax.axis_index("s") * _NC + lax.axis_index("c")
    lo = wid * _BLK_PER_W + jnp.minimum(wid, _BLK_EXTRA)
    hi = lo + _BLK_PER_W + jnp.where(wid < _BLK_EXTRA, 1, 0)

    iota = lax.iota(jnp.int32, _LANES)
    colsbase = lax.rem(iota, 4) * EMBED_DIM
    rows_g = [16 * g // 4 + iota // 4 for g in range(_TW // _LANES)]

    def issue_in(b, buf):
        return pltpu.async_copy(
            table_t_hbm.at[:, pl.ds(b * _TW, _TW)], vblk.at[buf], isem
        )

    def issue_out(b, buf):
        return pltpu.async_copy(
            tblk.at[buf], out_hbm.at[pl.ds(b * (_TW // 4), _TW // 4)], osem
        )

    def compute(buf):
        @plsc.parallel_loop(0, EMBED_DIM, unroll=2)
        def drow(d):
            cols_d = colsbase + d
            for g in range(_TW // _LANES):
                vals = vblk[buf, d, pl.ds(16 * g, 16)]
                plsc.store_scatter(tblk.at[buf], [rows_g[g], cols_d], vals)

    issue_in(lo, 0)

    def body(k, carry):
        buf = lax.rem(k - lo, 2)
        pltpu.make_async_copy(
            table_t_hbm.at[:, pl.ds(k * _TW, _TW)], vblk.at[buf], isem
        ).wait()

        @pl.when(k + 1 < hi)
        def _():
            issue_in(k + 1, lax.rem(k + 1 - lo, 2))

        @pl.when(k - 2 >= lo)
        def _():
            pltpu.make_async_copy(
                tblk.at[buf],
                out_hbm.at[pl.ds((k - 2) * (_TW // 4), _TW // 4)],
                osem,
            ).wait()

        compute(buf)
        issue_out(k, buf)
        return carry

    lax.fori_loop(lo, hi, body, 0)

    def drain(k, carry):
        pltpu.make_async_copy(
            tblk.at[lax.rem(k - lo, 2)],
            out_hbm.at[pl.ds(k * (_TW // 4), _TW // 4)],
            osem,
        ).wait()
        return carry

    lax.fori_loop(jnp.maximum(lo, hi - 2), hi, drain, 0)

    @pl.when(wid == 0)
    def _():
        pltpu.sync_copy(tail_hbm, tblk.at[0, pl.ds(0, _NBLK_REM // 4)])
        pltpu.sync_copy(
            tblk.at[0, pl.ds(0, _NBLK_REM // 4)],
            out_hbm.at[pl.ds(_NBLK * (_TW // 4), _NBLK_REM // 4)],
        )


_BPW = _B // _NW
_CB = 32
_CHUNK = _CB * FIELDS
_NCHUNK = _BPW // _CHUNK
_BATCH_PER_W = BATCH // _NW


@functools.partial(
    pl.kernel,
    mesh=_mesh,
    compiler_params=pltpu.CompilerParams(
        use_tc_tiling_on_sc=False, needs_layout_passes=False
    ),
    out_type=jax.ShapeDtypeStruct((FIELDS, EMBED_DIM, BATCH), jnp.float32),
    scratch_types=[
        pltpu.VMEM((_BPW,), jnp.int32),
        pltpu.VMEM((2, _CHUNK, EMBED_DIM), jnp.float32),
        pltpu.VMEM((2, EMBED_DIM, _CHUNK + 1), jnp.float32),
        pltpu.SemaphoreType.DMA,
        pltpu.SemaphoreType.DMA,
    ],
)
def _gather_kernel(idx_hbm, table_hbm, out_hbm, idx_v, rows_v, trans_v,
                   gsem, osem):
    wid = lax.axis_index("s") * _NC + lax.axis_index("c")
    b0 = wid * _BATCH_PER_W
    pltpu.sync_copy(idx_hbm.at[pl.ds(wid * _BPW, _BPW)], idx_v)

    iota = lax.iota(jnp.int32, _LANES)

    def issue_gather(c, buf):
        return pltpu.async_copy(
            table_hbm.at[idx_v.at[pl.ds(c * _CHUNK, _CHUNK)]],
            rows_v.at[buf], gsem,
        )

    def transpose_chunk(buf):
        @plsc.parallel_loop(0, _CHUNK, unroll=2)
        def jrow(j):
            jvec = jnp.full((_LANES,), j, jnp.int32)
            for g in range(EMBED_DIM // _LANES):
                vals = rows_v[buf, j, pl.ds(16 * g, 16)]
                plsc.store_scatter(
                    trans_v.at[buf], [16 * g + iota, jvec], vals
                )

    def issue_stores(c, buf):
        for f in range(FIELDS):
            pltpu.async_copy(
                trans_v.at[buf, :, pl.ds(f * _CB, _CB)],
                out_hbm.at[f, :, pl.ds(b0 + c * _CB, _CB)],
                osem,
            )

    def wait_stores(c, buf):
        for f in range(FIELDS):
            pltpu.make_async_copy(
                trans_v.at[buf, :, pl.ds(f * _CB, _CB)],
                out_hbm.at[f, :, pl.ds(b0 + c * _CB, _CB)],
                osem,
            ).wait()

    issue_gather(0, 0)

    def body(c, carry):
        buf = lax.rem(c, 2)
        pltpu.make_async_copy(
            table_hbm.at[idx_v.at[pl.ds(c * _CHUNK, _CHUNK)]],
            rows_v.at[buf], gsem,
        ).wait()

        @pl.when(c + 1 < _NCHUNK)
        def _():
            issue_gather(c + 1, lax.rem(c + 1, 2))

        @pl.when(c >= 2)
        def _():
            wait_stores(c - 2, buf)

        transpose_chunk(buf)
        issue_stores(c, buf)
        return carry

    lax.fori_loop(0, _NCHUNK, body, 0)
    wait_stores(_NCHUNK - 2, lax.rem(_NCHUNK - 2, 2))
    wait_stores(_NCHUNK - 1, lax.rem(_NCHUNK - 1, 2))


def kernel(z_category, categ_embed_weight):
    idx = (
        z_category.reshape(_NW, _NCHUNK, _CB, FIELDS)
        .transpose(0, 1, 3, 2)
        .reshape(-1)
        .astype(jnp.int32)
    )
    tail = categ_embed_weight[_NBLK * _TW:].reshape(_NBLK_REM // 4, 128)
    table_packed = _transpose_kernel(categ_embed_weight.T, tail)
    table_rm = table_packed.reshape(N_CLASS, EMBED_DIM)
    out_t = _gather_kernel(idx, table_rm)
    return out_t.transpose(2, 0, 1)


# speedup vs baseline: 1.2562x; 1.0001x over previous
import functools

import jax
import jax.numpy as jnp
from jax import lax
from jax.experimental import pallas as pl
from jax.experimental.pallas import tpu as pltpu
from jax.experimental.pallas import tpu_sc as plsc

N_CLASS = 1000000
EMBED_DIM = 32
BATCH = 16384
FIELDS = 26

_B = BATCH * FIELDS
_NC, _NS = 2, 16
_NW = _NC * _NS

_LANES = 16
_TW = 512
_NBLK = N_CLASS // _TW
_NBLK_REM = N_CLASS - _NBLK * _TW
_BLK_PER_W = _NBLK // _NW
_BLK_EXTRA = _NBLK % _NW

_mesh = plsc.VectorSubcoreMesh(core_axis_name="c", subcore_axis_name="s")


@functools.partial(
    pl.kernel,
    mesh=_mesh,
    compiler_params=pltpu.CompilerParams(
        use_tc_tiling_on_sc=True, needs_layout_passes=False
    ),
    out_type=jax.ShapeDtypeStruct((N_CLASS // 4, 128), jnp.float32),
    scratch_types=[
        pltpu.VMEM((2, EMBED_DIM, _TW), jnp.float32),
        pltpu.VMEM((2, _TW // 4, 133), jnp.float32),
        pltpu.SemaphoreType.DMA,
        pltpu.SemaphoreType.DMA,
    ],
)
def _transpose_kernel(table_t_hbm, tail_hbm, out_hbm, vblk, tblk, isem, osem):
    wid = lax.axis_index("s") * _NC + lax.axis_index("c")
    lo = wid * _BLK_PER_W + jnp.minimum(wid, _BLK_EXTRA)
    hi = lo + _BLK_PER_W + jnp.where(wid < _BLK_EXTRA, 1, 0)

    iota = lax.iota(jnp.int32, _LANES)
    colsbase = lax.rem(iota, 4) * EMBED_DIM
    rows_g = [16 * g // 4 + iota // 4 for g in range(_TW // _LANES)]

    def issue_in(b, buf):
        return pltpu.async_copy(
            table_t_hbm.at[:, pl.ds(b * _TW, _TW)], vblk.at[buf], isem
        )

    def issue_out(b, buf):
        return pltpu.async_copy(
            tblk.at[buf, :, pl.ds(0, 128)],
            out_hbm.at[pl.ds(b * (_TW // 4), _TW // 4)], osem,
        )

    def compute(buf):
        @plsc.parallel_loop(0, EMBED_DIM, unroll=2)
        def drow(d):
            cols_d = colsbase + d
            for g in range(_TW // _LANES):
                vals = vblk[buf, d, pl.ds(16 * g, 16)]
                plsc.store_scatter(tblk.at[buf], [rows_g[g], cols_d], vals)

    issue_in(lo, 0)

    def body(k, carry):
        buf = lax.rem(k - lo, 2)
        pltpu.make_async_copy(
            table_t_hbm.at[:, pl.ds(k * _TW, _TW)], vblk.at[buf], isem
        ).wait()

        @pl.when(k + 1 < hi)
        def _():
            issue_in(k + 1, lax.rem(k + 1 - lo, 2))

        @pl.when(k - 2 >= lo)
        def _():
            pltpu.make_async_copy(
                tblk.at[buf, :, pl.ds(0, 128)],
                out_hbm.at[pl.ds((k - 2) * (_TW // 4), _TW // 4)],
                osem,
            ).wait()

        compute(buf)
        issue_out(k, buf)
        return carry

    lax.fori_loop(lo, hi, body, 0)

    def drain(k, carry):
        pltpu.make_async_copy(
            tblk.at[lax.rem(k - lo, 2), :, pl.ds(0, 128)],
            out_hbm.at[pl.ds(k * (_TW // 4), _TW // 4)],
            osem,
        ).wait()
        return carry

    lax.fori_loop(jnp.maximum(lo, hi - 2), hi, drain, 0)

    @pl.when(wid == 0)
    def _():
        pltpu.sync_copy(
            tail_hbm, tblk.at[0, pl.ds(0, _NBLK_REM // 4), pl.ds(0, 128)]
        )
        pltpu.sync_copy(
            tblk.at[0, pl.ds(0, _NBLK_REM // 4), pl.ds(0, 128)],
            out_hbm.at[pl.ds(_NBLK * (_TW // 4), _NBLK_REM // 4)],
        )


_BPW = _B // _NW
_CB = 32
_CHUNK = _CB * FIELDS
_NCHUNK = _BPW // _CHUNK
_BATCH_PER_W = BATCH // _NW


@functools.partial(
    pl.kernel,
    mesh=_mesh,
    compiler_params=pltpu.CompilerParams(
        use_tc_tiling_on_sc=False, needs_layout_passes=False
    ),
    out_type=jax.ShapeDtypeStruct((FIELDS, EMBED_DIM, BATCH), jnp.float32),
    scratch_types=[
        pltpu.VMEM((_BPW,), jnp.int32),
        pltpu.VMEM((2, _CHUNK, EMBED_DIM), jnp.float32),
        pltpu.VMEM((2, EMBED_DIM, _CHUNK + 1), jnp.float32),
        pltpu.SemaphoreType.DMA,
        pltpu.SemaphoreType.DMA,
    ],
)
def _gather_kernel(idx_hbm, table_hbm, out_hbm, idx_v, rows_v, trans_v,
                   gsem, osem):
    wid = lax.axis_index("s") * _NC + lax.axis_index("c")
    b0 = wid * _BATCH_PER_W
    pltpu.sync_copy(idx_hbm.at[pl.ds(wid * _BPW, _BPW)], idx_v)

    iota = lax.iota(jnp.int32, _LANES)

    def issue_gather(c, buf):
        return pltpu.async_copy(
            table_hbm.at[idx_v.at[pl.ds(c * _CHUNK, _CHUNK)]],
            rows_v.at[buf], gsem,
        )

    def transpose_chunk(buf):
        @plsc.parallel_loop(0, _CHUNK, unroll=2)
        def jrow(j):
            jvec = jnp.full((_LANES,), j, jnp.int32)
            for g in range(EMBED_DIM // _LANES):
                vals = rows_v[buf, j, pl.ds(16 * g, 16)]
                plsc.store_scatter(
                    trans_v.at[buf], [16 * g + iota, jvec], vals
                )

    def issue_stores(c, buf):
        for f in range(FIELDS):
            pltpu.async_copy(
                trans_v.at[buf, :, pl.ds(f * _CB, _CB)],
                out_hbm.at[f, :, pl.ds(b0 + c * _CB, _CB)],
                osem,
            )

    def wait_stores(c, buf):
        for f in range(FIELDS):
            pltpu.make_async_copy(
                trans_v.at[buf, :, pl.ds(f * _CB, _CB)],
                out_hbm.at[f, :, pl.ds(b0 + c * _CB, _CB)],
                osem,
            ).wait()

    issue_gather(0, 0)

    def body(c, carry):
        buf = lax.rem(c, 2)
        pltpu.make_async_copy(
            table_hbm.at[idx_v.at[pl.ds(c * _CHUNK, _CHUNK)]],
            rows_v.at[buf], gsem,
        ).wait()

        @pl.when(c + 1 < _NCHUNK)
        def _():
            issue_gather(c + 1, lax.rem(c + 1, 2))

        @pl.when(c >= 2)
        def _():
            wait_stores(c - 2, buf)

        transpose_chunk(buf)
        issue_stores(c, buf)
        return carry

    lax.fori_loop(0, _NCHUNK, body, 0)
    wait_stores(_NCHUNK - 2, lax.rem(_NCHUNK - 2, 2))
    wait_stores(_NCHUNK - 1, lax.rem(_NCHUNK - 1, 2))


def kernel(z_category, categ_embed_weight):
    idx = (
        z_category.reshape(_NW, _NCHUNK, _CB, FIELDS)
        .transpose(0, 1, 3, 2)
        .reshape(-1)
        .astype(jnp.int32)
    )
    tail = categ_embed_weight[_NBLK * _TW:].reshape(_NBLK_REM // 4, 128)
    table_packed = _transpose_kernel(categ_embed_weight.T, tail)
    table_rm = table_packed.reshape(N_CLASS, EMBED_DIM)
    out_t = _gather_kernel(idx, table_rm)
    return out_t.transpose(2, 0, 1)
